# baseline (speedup 1.0000x reference)
import functools

import jax
import jax.numpy as jnp
from jax import lax
from jax.experimental import pallas as pl
from jax.experimental.pallas import tpu as pltpu
from jax.experimental.pallas import tpu_sc as plsc

_NC = 2
_NS = 16
_L = 16
_NW = _NC * _NS


@functools.lru_cache(maxsize=None)
def _make_mf(B, D, U):
    BPW = B // _NW
    C = 128
    NCHUNK = BPW // C
    NG = C // _L

    mesh = plsc.VectorSubcoreMesh(core_axis_name="c", subcore_axis_name="s")

    @functools.partial(
        pl.kernel,
        mesh=mesh,
        compiler_params=pltpu.CompilerParams(needs_layout_passes=False),
        out_type=jax.ShapeDtypeStruct((B,), jnp.float32),
        scratch_types=[
            pltpu.VMEM((BPW,), jnp.int32),
            pltpu.VMEM((BPW,), jnp.int32),
            pltpu.VMEM((C, D), jnp.float32),
            pltpu.VMEM((C, D), jnp.float32),
            pltpu.VMEM((BPW,), jnp.float32),
            pltpu.VMEM((BPW,), jnp.float32),
            pltpu.VMEM((BPW,), jnp.float32),
            pltpu.VMEM((_L,), jnp.float32),
            pltpu.VMEM((_L * _L,), jnp.float32),
            pltpu.SemaphoreType.DMA,
        ],
    )
    def mf(uid_hbm, aid_hbm, uemb_hbm, aemb_hbm, ubias_hbm, abias_hbm,
           gb_hbm, out_hbm, uid_v, aid_v, u_v, a_v, ub_v, ab_v, out_v,
           gb_v, tbuf, sem):
        wid = lax.axis_index("s") * _NC + lax.axis_index("c")
        base = wid * BPW
        pltpu.sync_copy(uid_hbm.at[pl.ds(base, BPW)], uid_v)
        pltpu.sync_copy(aid_hbm.at[pl.ds(base, BPW)], aid_v)
        pltpu.sync_copy(gb_hbm, gb_v)

        lanes = lax.iota(jnp.int32, _L)
        zeros16 = lanes * 0
        gvec = gb_v[...]

        for chunk in range(NCHUNK):
            cb = chunk * C
            idx_u = uid_v.at[pl.ds(cb, C)]
            idx_a = aid_v.at[pl.ds(cb, C)]
            cu = pltpu.async_copy(uemb_hbm.at[idx_u], u_v, sem)
            ca = pltpu.async_copy(aemb_hbm.at[idx_a], a_v, sem)
            cub = pltpu.async_copy(ubias_hbm.at[idx_u], ub_v.at[pl.ds(cb, C)], sem)
            cab = pltpu.async_copy(abias_hbm.at[idx_a], ab_v.at[pl.ds(cb, C)], sem)
            cu.wait()
            ca.wait()
            cub.wait()
            cab.wait()

            def group_body(g, _, cb=cb):
                rowbase = g * _L
                for rr in range(_L):
                    r = rowbase + rr
                    acc = u_v[r, pl.ds(0, _L)] * a_v[r, pl.ds(0, _L)]
                    for k in range(1, D // _L):
                        acc = acc + (u_v[r, pl.ds(k * _L, _L)]
                                     * a_v[r, pl.ds(k * _L, _L)])
                    tbuf[pl.ds(rr * _L, _L)] = acc
                dots = plsc.load_gather(tbuf, [lanes * _L])
                for c in range(1, _L):
                    dots = dots + plsc.load_gather(tbuf, [lanes * _L + c])
                off = cb + rowbase
                res = dots + ub_v[pl.ds(off, _L)] + ab_v[pl.ds(off, _L)] + gvec
                out_v[pl.ds(off, _L)] = res
                return 0

            lax.fori_loop(0, NG, group_body, 0)

        pltpu.sync_copy(out_v, out_hbm.at[pl.ds(base, BPW)])

    return mf


def kernel(user_id, anime_id, user_embedding, anime_embedding, user_bias,
           anime_bias, global_bias):
    B = user_id.shape[0]
    U, D = user_embedding.shape
    mf = _make_mf(B, D, U)
    return mf(
        user_id.astype(jnp.int32),
        anime_id.astype(jnp.int32),
        user_embedding,
        anime_embedding,
        user_bias.reshape(-1),
        anime_bias.reshape(-1),
        jnp.broadcast_to(global_bias, (_L,)),
    )


# speedup vs baseline: 1.0571x; 1.0571x over previous
import functools

import jax
import jax.numpy as jnp
from jax import lax
from jax.experimental import pallas as pl
from jax.experimental.pallas import tpu as pltpu
from jax.experimental.pallas import tpu_sc as plsc

_NC = 2
_NS = 16
_L = 16
_NW = _NC * _NS


@functools.lru_cache(maxsize=None)
def _make_mf(B, D, U):
    BPW = B // _NW
    C = 128
    NCHUNK = BPW // C
    NG = C // _L

    mesh = plsc.VectorSubcoreMesh(core_axis_name="c", subcore_axis_name="s")

    @functools.partial(
        pl.kernel,
        mesh=mesh,
        compiler_params=pltpu.CompilerParams(needs_layout_passes=False),
        out_type=jax.ShapeDtypeStruct((B,), jnp.float32),
        scratch_types=[
            pltpu.VMEM((BPW,), jnp.int32),
            pltpu.VMEM((BPW,), jnp.int32),
            pltpu.VMEM((2, C, D), jnp.float32),
            pltpu.VMEM((2, C, D), jnp.float32),
            pltpu.VMEM((BPW,), jnp.float32),
            pltpu.VMEM((BPW,), jnp.float32),
            pltpu.VMEM((BPW,), jnp.float32),
            pltpu.VMEM((_L,), jnp.float32),
            pltpu.VMEM((_L * _L,), jnp.float32),
            pltpu.SemaphoreType.DMA,
            pltpu.SemaphoreType.DMA,
            pltpu.SemaphoreType.DMA,
        ],
    )
    def mf(uid_hbm, aid_hbm, uemb_hbm, aemb_hbm, ubias_hbm, abias_hbm,
           gb_hbm, out_hbm, uid_v, aid_v, u_v, a_v, ub_v, ab_v, out_v,
           gb_v, tbuf, sem0, sem1, semb):
        wid = lax.axis_index("s") * _NC + lax.axis_index("c")
        base = wid * BPW
        pltpu.sync_copy(uid_hbm.at[pl.ds(base, BPW)], uid_v)
        pltpu.sync_copy(aid_hbm.at[pl.ds(base, BPW)], aid_v)
        pltpu.sync_copy(gb_hbm, gb_v)

        lanes = lax.iota(jnp.int32, _L)
        zeros16 = lanes * 0
        gvec = gb_v[...]

        sems = (sem0, sem1)

        def start_rows(chunk):
            b = chunk % 2
            cb = chunk * C
            du = pltpu.async_copy(
                uemb_hbm.at[uid_v.at[pl.ds(cb, C)]], u_v.at[b], sems[b])
            da = pltpu.async_copy(
                aemb_hbm.at[aid_v.at[pl.ds(cb, C)]], a_v.at[b], sems[b])
            return du, da

        row_descs = [None] * NCHUNK
        row_descs[0] = start_rows(0)
        bias_descs = []
        for chunk in range(NCHUNK):
            cb = chunk * C
            bias_descs.append(pltpu.async_copy(
                ubias_hbm.at[uid_v.at[pl.ds(cb, C)]],
                ub_v.at[pl.ds(cb, C)], semb))
            bias_descs.append(pltpu.async_copy(
                abias_hbm.at[aid_v.at[pl.ds(cb, C)]],
                ab_v.at[pl.ds(cb, C)], semb))
        if NCHUNK > 1:
            row_descs[1] = start_rows(1)
        for d in bias_descs:
            d.wait()

        for chunk in range(NCHUNK):
            buf = chunk % 2
            cb = chunk * C
            du, da = row_descs[chunk]
            du.wait()
            da.wait()

            def group_body(g, _, cb=cb, buf=buf):
                rowbase = g * _L
                for rr in range(_L):
                    r = rowbase + rr
                    acc = u_v[buf, r, pl.ds(0, _L)] * a_v[buf, r, pl.ds(0, _L)]
                    for k in range(1, D // _L):
                        acc = acc + (u_v[buf, r, pl.ds(k * _L, _L)]
                                     * a_v[buf, r, pl.ds(k * _L, _L)])
                    tbuf[pl.ds(rr * _L, _L)] = acc
                dots = plsc.load_gather(tbuf, [lanes * _L])
                for c in range(1, _L):
                    dots = dots + plsc.load_gather(tbuf, [lanes * _L + c])
                off = cb + rowbase
                res = dots + ub_v[pl.ds(off, _L)] + ab_v[pl.ds(off, _L)] + gvec
                out_v[pl.ds(off, _L)] = res
                return 0

            lax.fori_loop(0, NG, group_body, 0)

            if chunk + 2 < NCHUNK:
                row_descs[chunk + 2] = start_rows(chunk + 2)

        pltpu.sync_copy(out_v, out_hbm.at[pl.ds(base, BPW)])

    return mf


def kernel(user_id, anime_id, user_embedding, anime_embedding, user_bias,
           anime_bias, global_bias):
    B = user_id.shape[0]
    U, D = user_embedding.shape
    mf = _make_mf(B, D, U)
    return mf(
        user_id.astype(jnp.int32),
        anime_id.astype(jnp.int32),
        user_embedding,
        anime_embedding,
        user_bias.reshape(-1),
        anime_bias.reshape(-1),
        jnp.broadcast_to(global_bias, (_L,)),
    )


# speedup vs baseline: 1.9438x; 1.8388x over previous
import functools

import jax
import jax.numpy as jnp
from jax import lax
from jax.experimental import pallas as pl
from jax.experimental.pallas import tpu as pltpu
from jax.experimental.pallas import tpu_sc as plsc

_NC = 2
_NS = 16
_L = 16
_NW = _NC * _NS


@functools.lru_cache(maxsize=None)
def _make_mf(B, D, UP, AP):
    BPW = B // _NW
    C = 128
    NCHUNK = BPW // C
    NG = C // _L

    mesh = plsc.VectorSubcoreMesh(core_axis_name="c", subcore_axis_name="s")

    @functools.partial(
        pl.kernel,
        mesh=mesh,
        compiler_params=pltpu.CompilerParams(needs_layout_passes=False),
        out_type=jax.ShapeDtypeStruct((B,), jnp.float32),
        scratch_types=[
            pltpu.VMEM((BPW,), jnp.int32),
            pltpu.VMEM((BPW,), jnp.int32),
            pltpu.VMEM((2, C, D), jnp.float32),
            pltpu.VMEM((2, C, D), jnp.float32),
            pltpu.VMEM((BPW,), jnp.float32),
            pltpu.VMEM((BPW,), jnp.float32),
            pltpu.VMEM((BPW,), jnp.float32),
            pltpu.VMEM((_L,), jnp.float32),
            pltpu.VMEM((_L * _L,), jnp.float32),
            pltpu.SemaphoreType.DMA,
            pltpu.SemaphoreType.DMA,
            pltpu.SemaphoreType.DMA,
        ],
    )
    def mf(uid_hbm, aid_hbm, uemb_hbm, aemb_hbm, ubias_hbm, abias_hbm,
           gb_hbm, out_hbm, uid_v, aid_v, u_v, a_v, ub_v, ab_v, out_v,
           gb_v, tbuf, sem0, sem1, semb):
        wid = lax.axis_index("s") * _NC + lax.axis_index("c")
        base = wid * BPW
        pltpu.sync_copy(uid_hbm.at[pl.ds(base, BPW)], uid_v)
        pltpu.sync_copy(aid_hbm.at[pl.ds(base, BPW)], aid_v)
        pltpu.sync_copy(gb_hbm.at[pl.ds(0, 1)], gb_v.at[pl.ds(0, 1)])

        lanes = lax.iota(jnp.int32, _L)
        zeros16 = lanes * 0
        gvec = plsc.load_gather(gb_v, [zeros16])

        sems = (sem0, sem1)

        def start_rows(chunk):
            b = chunk % 2
            cb = chunk * C
            du = pltpu.async_copy(
                uemb_hbm.at[uid_v.at[pl.ds(cb, C)]], u_v.at[b], sems[b])
            da = pltpu.async_copy(
                aemb_hbm.at[aid_v.at[pl.ds(cb, C)]], a_v.at[b], sems[b])
            return du, da

        row_descs = [None] * NCHUNK
        row_descs[0] = start_rows(0)
        bias_descs = []
        for chunk in range(NCHUNK):
            cb = chunk * C
            bias_descs.append(pltpu.async_copy(
                ubias_hbm.at[uid_v.at[pl.ds(cb, C)]],
                ub_v.at[pl.ds(cb, C)], semb))
            bias_descs.append(pltpu.async_copy(
                abias_hbm.at[aid_v.at[pl.ds(cb, C)]],
                ab_v.at[pl.ds(cb, C)], semb))
        if NCHUNK > 1:
            row_descs[1] = start_rows(1)
        for d in bias_descs:
            d.wait()

        for chunk in range(NCHUNK):
            buf = chunk % 2
            cb = chunk * C
            du, da = row_descs[chunk]
            du.wait()
            da.wait()

            def group_body(g, _, cb=cb, buf=buf):
                rowbase = g * _L
                for rr in range(_L):
                    r = rowbase + rr
                    acc = u_v[buf, r, pl.ds(0, _L)] * a_v[buf, r, pl.ds(0, _L)]
                    for k in range(1, D // _L):
                        acc = acc + (u_v[buf, r, pl.ds(k * _L, _L)]
                                     * a_v[buf, r, pl.ds(k * _L, _L)])
                    tbuf[pl.ds(rr * _L, _L)] = acc
                dots = plsc.load_gather(tbuf, [lanes * _L])
                for c in range(1, _L):
                    dots = dots + plsc.load_gather(tbuf, [lanes * _L + c])
                off = cb + rowbase
                res = dots + ub_v[pl.ds(off, _L)] + ab_v[pl.ds(off, _L)] + gvec
                out_v[pl.ds(off, _L)] = res
                return 0

            lax.fori_loop(0, NG, group_body, 0)

            if chunk + 2 < NCHUNK:
                row_descs[chunk + 2] = start_rows(chunk + 2)

        pltpu.sync_copy(out_v, out_hbm.at[pl.ds(base, BPW)])

    return mf


def _flat_pad(bias):
    n = bias.shape[0]
    pad = (-n) % 1024
    return jnp.concatenate(
        [bias, jnp.zeros((pad, 1), bias.dtype)], axis=0).reshape(-1)


def kernel(user_id, anime_id, user_embedding, anime_embedding, user_bias,
           anime_bias, global_bias):
    B = user_id.shape[0]
    U, D = user_embedding.shape
    ub_flat = _flat_pad(user_bias)
    ab_flat = _flat_pad(anime_bias)
    mf = _make_mf(B, D, ub_flat.shape[0], ab_flat.shape[0])
    return mf(
        user_id.astype(jnp.int32),
        anime_id.astype(jnp.int32),
        user_embedding,
        anime_embedding,
        ub_flat,
        ab_flat,
        global_bias,
    )
